# baseline (speedup 1.0000x reference)
import dataclasses
import functools

import jax
import jax.numpy as jnp
from jax import lax
from jax.experimental import pallas as pl
from jax.experimental.pallas import tpu as pltpu
from jax.experimental.pallas import tpu_sc as plsc

NROW = 32
NCOL = 1_000_000
NSAMP = 64
BC = 160
NB = NCOL // BC
NBP = 6400
L = 16
CHUNK = 8000


def _bsum_body(x_ref, o_ref):
    o_ref[...] = jnp.sum(x_ref[...], axis=-1, keepdims=True)


def _block_sums(x2):
    return pl.pallas_call(
        _bsum_body,
        grid=(NROW * NB // CHUNK,),
        in_specs=[pl.BlockSpec((CHUNK, BC), lambda i: (i, 0))],
        out_specs=pl.BlockSpec((CHUNK, 1), lambda i: (i, 0)),
        out_shape=jax.ShapeDtypeStruct((NROW * NB, 1), jnp.float32),
    )(x2)


def _sc_compiler_params():
    cp = pltpu.CompilerParams()
    if "needs_layout_passes" in pltpu.CompilerParams.__dataclass_fields__:
        cp = dataclasses.replace(cp, needs_layout_passes=False)
    return cp


def _sample_body(x_hbm, bs_hbm, u_hbm, o_hbm, bs_v, bcum_v, t_v, blk_v,
                 res_v, sem):
    del sem
    r = lax.axis_index("s") * 2 + lax.axis_index("c")
    iota = lax.iota(jnp.int32, L)

    pltpu.sync_copy(bs_hbm.at[r], bs_v)

    def cum_body(i, carry):
        v = bs_v[pl.ds(i * L, L)]
        c = plsc.cumsum(v) + carry
        bcum_v[pl.ds(i * L, L)] = c
        return jnp.max(c)

    total = lax.fori_loop(0, NBP // L, cum_body, jnp.float32(0.0))

    pltpu.sync_copy(u_hbm.at[r], t_v)
    for j in range(NSAMP // L):
        t_v[pl.ds(j * L, L)] = t_v[pl.ds(j * L, L)] * total

    def sample_body(k, acc):
        kk = jnp.full((L,), k, jnp.int32)
        tb = plsc.load_gather(t_v, [kk])

        lo = jnp.int32(0)
        basev = jnp.zeros((L,), jnp.float32)
        for s in (512, 32, 2, 1):
            p = jnp.minimum(lo + (iota + 1) * s - 1, NBP - 1)
            v = plsc.load_gather(bcum_v, [p])
            le = v <= tb
            cnt = jnp.sum(le.astype(jnp.int32))
            basev = jnp.maximum(basev, jnp.where(le, v, 0.0))
            lo = lo + cnt * s

        block = jnp.minimum(lo, NB - 1)
        base = jnp.max(basev)

        pltpu.sync_copy(x_hbm.at[pl.ds(r * NCOL + block * BC, BC)], blk_v)

        cnt2 = jnp.int32(0)
        carry = base
        for g in range(BC // L):
            v = blk_v[pl.ds(g * L, L)]
            cs = plsc.cumsum(v) + carry
            cnt2 = cnt2 + jnp.sum((cs <= tb).astype(jnp.int32))
            carry = jnp.max(cs)

        final = jnp.minimum(block * BC + cnt2, NCOL - 1)
        plsc.store_scatter(res_v, [kk], jnp.full((L,), final, jnp.int32),
                           mask=iota == 0)
        return acc

    lax.fori_loop(0, NSAMP, sample_body, jnp.int32(0))
    pltpu.sync_copy(res_v, o_hbm.at[r])


@functools.partial(
    pl.kernel,
    out_type=jax.ShapeDtypeStruct((NROW, NSAMP), jnp.int32),
    mesh=plsc.VectorSubcoreMesh(core_axis_name="c", subcore_axis_name="s"),
    scratch_types=[
        pltpu.VMEM((NBP,), jnp.float32),
        pltpu.VMEM((NBP,), jnp.float32),
        pltpu.VMEM((NSAMP,), jnp.float32),
        pltpu.VMEM((BC,), jnp.float32),
        pltpu.VMEM((NSAMP,), jnp.int32),
        pltpu.SemaphoreType.DMA,
    ],
    compiler_params=_sc_compiler_params(),
)
def _sample_kernel(x_hbm, bs_hbm, u_hbm, o_hbm, bs_v, bcum_v, t_v, blk_v,
                   res_v, sem):
    _sample_body(x_hbm, bs_hbm, u_hbm, o_hbm, bs_v, bcum_v, t_v, blk_v,
                 res_v, sem)


def kernel(x):
    x2 = x.reshape(NROW * NB, BC)
    bsums = _block_sums(x2).reshape(NROW, NB)
    bs_pad = jnp.pad(bsums, ((0, 0), (0, NBP - NB)))
    u = jax.random.uniform(jax.random.key(42), (NROW, NSAMP),
                           dtype=jnp.float32)
    idx = _sample_kernel(x.reshape(-1), bs_pad, u)
    return idx.astype(jnp.int64)


# speedup vs baseline: 11.1407x; 11.1407x over previous
import dataclasses
import functools

import jax
import jax.numpy as jnp
from jax import lax
from jax.experimental import pallas as pl
from jax.experimental.pallas import tpu as pltpu
from jax.experimental.pallas import tpu_sc as plsc

NROW = 32
NCOL = 1_000_000
NSAMP = 64
BC = 128
CT = 128
CC = CT * BC
NSTEP = -(-NCOL // CC)
NG = NROW // 8
GI = NG * NSTEP
NB = -(-NCOL // BC)
NBP = NSTEP * CT
L = 16


def _pass1_body(x_ref, copy_ref, bs_ref):
    i = pl.program_id(1)
    col = (i * CC
           + lax.broadcasted_iota(jnp.int32, (8, CT, BC), 1) * BC
           + lax.broadcasted_iota(jnp.int32, (8, CT, BC), 2))
    xm = jnp.where(col < NCOL, x_ref[...].reshape(8, CT, BC), 0.0)
    bs_ref[0] = jnp.sum(xm, axis=-1)
    for ct in range(CT):
        copy_ref[0, ct] = xm[:, ct, :]


def _pass1(x):
    return pl.pallas_call(
        _pass1_body,
        grid=(NG, NSTEP),
        in_specs=[pl.BlockSpec((8, CC), lambda g, i: (g, i))],
        out_specs=[
            pl.BlockSpec((1, CT, 8, BC), lambda g, i: (g * NSTEP + i, 0, 0, 0)),
            pl.BlockSpec((1, 8, CT), lambda g, i: (g * NSTEP + i, 0, 0)),
        ],
        out_shape=[
            jax.ShapeDtypeStruct((GI, CT, 8, BC), jnp.float32),
            jax.ShapeDtypeStruct((GI, 8, CT), jnp.float32),
        ],
    )(x)


def _sc_compiler_params():
    cp = pltpu.CompilerParams()
    if "needs_layout_passes" in pltpu.CompilerParams.__dataclass_fields__:
        cp = dataclasses.replace(cp, needs_layout_passes=False)
    return cp


def _sample_body(xl_hbm, bs_hbm, u_hbm, o_hbm, bs_v, bcum_v, t_v, blk_v,
                 res_v, sem):
    r = lax.axis_index("s") * 2 + lax.axis_index("c")
    g = r // 8
    s = r % 8
    iota = lax.iota(jnp.int32, L)

    copies = [
        pltpu.async_copy(bs_hbm.at[g * NSTEP + i, s],
                         bs_v.at[pl.ds(i * CT, CT)], sem)
        for i in range(NSTEP)
    ]
    for cp in copies:
        cp.wait()

    def cum_body(i, carry):
        c = plsc.cumsum(bs_v[pl.ds(i * L, L)]) + carry
        bcum_v[pl.ds(i * L, L)] = c
        return jnp.max(c)

    total = lax.fori_loop(0, NBP // L, cum_body, jnp.float32(0.0))

    pltpu.sync_copy(u_hbm.at[r], t_v)
    for j in range(NSAMP // L):
        t_v[pl.ds(j * L, L)] = t_v[pl.ds(j * L, L)] * total

    def sample_body(k, acc):
        kk = jnp.full((L,), k, jnp.int32)
        tb = plsc.load_gather(t_v, [kk])

        lo = jnp.int32(0)
        basev = jnp.zeros((L,), jnp.float32)
        for st in (512, 32, 2, 1):
            p = jnp.minimum(lo + (iota + 1) * st - 1, NBP - 1)
            v = plsc.load_gather(bcum_v, [p])
            le = v <= tb
            cnt = jnp.sum(le.astype(jnp.int32))
            basev = jnp.maximum(basev, jnp.where(le, v, 0.0))
            lo = lo + cnt * st

        block = jnp.minimum(lo, NB - 1)
        base = jnp.max(basev)

        pltpu.sync_copy(xl_hbm.at[g * NSTEP + block // CT, block % CT, s],
                        blk_v)

        cnt2 = jnp.int32(0)
        carry = base
        for gg in range(BC // L):
            cs = plsc.cumsum(blk_v[pl.ds(gg * L, L)]) + carry
            cnt2 = cnt2 + jnp.sum((cs <= tb).astype(jnp.int32))
            carry = jnp.max(cs)

        final = jnp.minimum(block * BC + cnt2, NCOL - 1)
        plsc.store_scatter(res_v, [kk], jnp.full((L,), final, jnp.int32),
                           mask=iota == 0)
        return acc

    lax.fori_loop(0, NSAMP, sample_body, jnp.int32(0))
    pltpu.sync_copy(res_v, o_hbm.at[r])


@functools.partial(
    pl.kernel,
    out_type=jax.ShapeDtypeStruct((NROW, NSAMP), jnp.int32),
    mesh=plsc.VectorSubcoreMesh(core_axis_name="c", subcore_axis_name="s"),
    scratch_types=[
        pltpu.VMEM((NBP,), jnp.float32),
        pltpu.VMEM((NBP,), jnp.float32),
        pltpu.VMEM((NSAMP,), jnp.float32),
        pltpu.VMEM((BC,), jnp.float32),
        pltpu.VMEM((NSAMP,), jnp.int32),
        pltpu.SemaphoreType.DMA,
    ],
    compiler_params=_sc_compiler_params(),
)
def _sample_kernel(xl_hbm, bs_hbm, u_hbm, o_hbm, bs_v, bcum_v, t_v, blk_v,
                   res_v, sem):
    _sample_body(xl_hbm, bs_hbm, u_hbm, o_hbm, bs_v, bcum_v, t_v, blk_v,
                 res_v, sem)


def kernel(x):
    xl, bs = _pass1(x)
    u = jax.random.uniform(jax.random.key(42), (NROW, NSAMP),
                           dtype=jnp.float32)
    idx = _sample_kernel(xl, bs, u)
    return idx.astype(jnp.int64)


# speedup vs baseline: 12.9614x; 1.1634x over previous
import dataclasses
import functools

import jax
import jax.numpy as jnp
from jax import lax
from jax.experimental import pallas as pl
from jax.experimental.pallas import tpu as pltpu
from jax.experimental.pallas import tpu_sc as plsc

NROW = 32
NCOL = 1_000_000
NSAMP = 64
BC = 128
CT = 128
CC = CT * BC
NSTEP = -(-NCOL // CC)
NG = NROW // 8
GI = NG * NSTEP
NB = -(-NCOL // BC)
NBP = NSTEP * CT
L = 16


def _pass1_body(x_ref, bs_ref):
    i = pl.program_id(1)
    col = (i * CC
           + lax.broadcasted_iota(jnp.int32, (8, CT, BC), 1) * BC
           + lax.broadcasted_iota(jnp.int32, (8, CT, BC), 2))
    xm = jnp.where(col < NCOL, x_ref[...].reshape(8, CT, BC), 0.0)
    bs_ref[0] = jnp.sum(xm, axis=-1)


def _pass1(x):
    return pl.pallas_call(
        _pass1_body,
        grid=(NG, NSTEP),
        in_specs=[pl.BlockSpec((8, CC), lambda g, i: (g, i))],
        out_specs=pl.BlockSpec((1, 8, CT), lambda g, i: (g * NSTEP + i, 0, 0)),
        out_shape=jax.ShapeDtypeStruct((GI, 8, CT), jnp.float32),
    )(x)


def _sc_compiler_params():
    cp = pltpu.CompilerParams(use_tc_tiling_on_sc=True)
    if "needs_layout_passes" in pltpu.CompilerParams.__dataclass_fields__:
        cp = dataclasses.replace(cp, needs_layout_passes=False)
    return cp


def _sample_body(xl_hbm, bs_hbm, u_hbm, o_hbm, bs_v, bcum_v, t_v, blk_v,
                 res_v, sem):
    r = lax.axis_index("s") * 2 + lax.axis_index("c")
    g = r // 8
    s = r % 8
    iota = lax.iota(jnp.int32, L)

    copies = [
        pltpu.async_copy(bs_hbm.at[g * NSTEP + i, s],
                         bs_v.at[pl.ds(i * CT, CT)], sem)
        for i in range(NSTEP)
    ]
    for cp in copies:
        cp.wait()

    def cum_body(i, carry):
        c = plsc.cumsum(bs_v[pl.ds(i * L, L)]) + carry
        bcum_v[pl.ds(i * L, L)] = c
        return jnp.max(c)

    total = lax.fori_loop(0, NBP // L, cum_body, jnp.float32(0.0))

    pltpu.sync_copy(u_hbm.at[r], t_v)
    for j in range(NSAMP // L):
        t_v[pl.ds(j * L, L)] = t_v[pl.ds(j * L, L)] * total

    def sample_body(k, acc):
        kk = jnp.full((L,), k, jnp.int32)
        tb = plsc.load_gather(t_v, [kk])

        lo = jnp.int32(0)
        basev = jnp.zeros((L,), jnp.float32)
        for st in (512, 32, 2, 1):
            p = jnp.minimum(lo + (iota + 1) * st - 1, NBP - 1)
            v = plsc.load_gather(bcum_v, [p])
            le = v <= tb
            cnt = jnp.sum(le.astype(jnp.int32))
            basev = jnp.maximum(basev, jnp.where(le, v, 0.0))
            lo = lo + cnt * st

        block = jnp.minimum(lo, NB - 1)
        base = jnp.max(basev)

        pltpu.sync_copy(xl_hbm.at[r, pl.ds(block * BC, BC)], blk_v)

        cnt2 = jnp.int32(0)
        carry = base
        for gg in range(BC // L):
            cs = plsc.cumsum(blk_v[pl.ds(gg * L, L)]) + carry
            cnt2 = cnt2 + jnp.sum((cs <= tb).astype(jnp.int32))
            carry = jnp.max(cs)

        final = jnp.minimum(block * BC + cnt2, NCOL - 1)
        plsc.store_scatter(res_v, [kk], jnp.full((L,), final, jnp.int32),
                           mask=iota == 0)
        return acc

    lax.fori_loop(0, NSAMP, sample_body, jnp.int32(0))
    pltpu.sync_copy(res_v, o_hbm.at[r])


@functools.partial(
    pl.kernel,
    out_type=jax.ShapeDtypeStruct((NROW, NSAMP), jnp.int32),
    mesh=plsc.VectorSubcoreMesh(core_axis_name="c", subcore_axis_name="s"),
    scratch_types=[
        pltpu.VMEM((NBP,), jnp.float32),
        pltpu.VMEM((NBP,), jnp.float32),
        pltpu.VMEM((NSAMP,), jnp.float32),
        pltpu.VMEM((BC,), jnp.float32),
        pltpu.VMEM((NSAMP,), jnp.int32),
        pltpu.SemaphoreType.DMA,
    ],
    compiler_params=_sc_compiler_params(),
)
def _sample_kernel(xl_hbm, bs_hbm, u_hbm, o_hbm, bs_v, bcum_v, t_v, blk_v,
                   res_v, sem):
    _sample_body(xl_hbm, bs_hbm, u_hbm, o_hbm, bs_v, bcum_v, t_v, blk_v,
                 res_v, sem)


def kernel(x):
    bs = _pass1(x)
    u = jax.random.uniform(jax.random.key(42), (NROW, NSAMP),
                           dtype=jnp.float32)
    idx = _sample_kernel(x, bs, u)
    return idx.astype(jnp.int64)


# speedup vs baseline: 20.1859x; 1.5574x over previous
import dataclasses
import functools

import jax
import jax.numpy as jnp
from jax import lax
from jax.experimental import pallas as pl
from jax.experimental.pallas import tpu as pltpu
from jax.experimental.pallas import tpu_sc as plsc

NROW = 32
NCOL = 1_000_000
NSAMP = 64
BC = 128
CT = 512
CC = CT * BC
NSTEP = -(-NCOL // CC)
NG = NROW // 8
GI = NG * NSTEP
NB = -(-NCOL // BC)
NBP = NSTEP * CT
L = 16


def _pass1_body(x_ref, bs_ref):
    i = pl.program_id(1)
    x3 = x_ref[...].reshape(8, CT, BC)

    @pl.when(i < NSTEP - 1)
    def _full():
        bs_ref[0] = jnp.sum(x3, axis=-1)

    @pl.when(i == NSTEP - 1)
    def _masked():
        col = (i * CC
               + lax.broadcasted_iota(jnp.int32, (8, CT, BC), 1) * BC
               + lax.broadcasted_iota(jnp.int32, (8, CT, BC), 2))
        bs_ref[0] = jnp.sum(jnp.where(col < NCOL, x3, 0.0), axis=-1)


def _pass1(x):
    return pl.pallas_call(
        _pass1_body,
        grid=(NG, NSTEP),
        in_specs=[pl.BlockSpec((8, CC), lambda g, i: (g, i))],
        out_specs=pl.BlockSpec((1, 8, CT), lambda g, i: (g * NSTEP + i, 0, 0)),
        out_shape=jax.ShapeDtypeStruct((GI, 8, CT), jnp.float32),
    )(x)


def _sc_compiler_params():
    cp = pltpu.CompilerParams(use_tc_tiling_on_sc=True)
    if "needs_layout_passes" in pltpu.CompilerParams.__dataclass_fields__:
        cp = dataclasses.replace(cp, needs_layout_passes=False)
    return cp


def _sample_body(xl_hbm, bs_hbm, u_hbm, o_hbm, bs_v, bcum_v, t_v, blk_v,
                 res_v, sem):
    r = lax.axis_index("s") * 2 + lax.axis_index("c")
    g = r // 8
    s = r % 8
    iota = lax.iota(jnp.int32, L)

    copies = [
        pltpu.async_copy(bs_hbm.at[g * NSTEP + i, s],
                         bs_v.at[pl.ds(i * CT, CT)], sem)
        for i in range(NSTEP)
    ]
    for cp in copies:
        cp.wait()

    def cum_body(i, carry):
        c = plsc.cumsum(bs_v[pl.ds(i * L, L)]) + carry
        bcum_v[pl.ds(i * L, L)] = c
        return jnp.max(c)

    total = lax.fori_loop(0, NBP // L, cum_body, jnp.float32(0.0))

    pltpu.sync_copy(u_hbm.at[r], t_v)
    for j in range(NSAMP // L):
        t_v[pl.ds(j * L, L)] = t_v[pl.ds(j * L, L)] * total

    def sample_body(k, acc):
        kk = jnp.full((L,), k, jnp.int32)
        tb = plsc.load_gather(t_v, [kk])

        lo = jnp.int32(0)
        basev = jnp.zeros((L,), jnp.float32)
        for st in (512, 32, 2, 1):
            p = jnp.minimum(lo + (iota + 1) * st - 1, NBP - 1)
            v = plsc.load_gather(bcum_v, [p])
            le = v <= tb
            cnt = jnp.sum(le.astype(jnp.int32))
            basev = jnp.maximum(basev, jnp.where(le, v, 0.0))
            lo = lo + cnt * st

        block = jnp.minimum(lo, NB - 1)
        base = jnp.max(basev)

        pltpu.sync_copy(xl_hbm.at[r, pl.ds(block * BC, BC)], blk_v)

        cnt2 = jnp.int32(0)
        carry = base
        for gg in range(BC // L):
            cs = plsc.cumsum(blk_v[pl.ds(gg * L, L)]) + carry
            cnt2 = cnt2 + jnp.sum((cs <= tb).astype(jnp.int32))
            carry = jnp.max(cs)

        final = jnp.minimum(block * BC + cnt2, NCOL - 1)
        plsc.store_scatter(res_v, [kk], jnp.full((L,), final, jnp.int32),
                           mask=iota == 0)
        return acc

    lax.fori_loop(0, NSAMP, sample_body, jnp.int32(0))
    pltpu.sync_copy(res_v, o_hbm.at[r])


@functools.partial(
    pl.kernel,
    out_type=jax.ShapeDtypeStruct((NROW, NSAMP), jnp.int32),
    mesh=plsc.VectorSubcoreMesh(core_axis_name="c", subcore_axis_name="s"),
    scratch_types=[
        pltpu.VMEM((NBP,), jnp.float32),
        pltpu.VMEM((NBP,), jnp.float32),
        pltpu.VMEM((NSAMP,), jnp.float32),
        pltpu.VMEM((BC,), jnp.float32),
        pltpu.VMEM((NSAMP,), jnp.int32),
        pltpu.SemaphoreType.DMA,
    ],
    compiler_params=_sc_compiler_params(),
)
def _sample_kernel(xl_hbm, bs_hbm, u_hbm, o_hbm, bs_v, bcum_v, t_v, blk_v,
                   res_v, sem):
    _sample_body(xl_hbm, bs_hbm, u_hbm, o_hbm, bs_v, bcum_v, t_v, blk_v,
                 res_v, sem)


def kernel(x):
    bs = _pass1(x)
    u = jax.random.uniform(jax.random.key(42), (NROW, NSAMP),
                           dtype=jnp.float32)
    idx = _sample_kernel(x, bs, u)
    return idx.astype(jnp.int64)


# speedup vs baseline: 26.5028x; 1.3129x over previous
import dataclasses
import functools

import jax
import jax.numpy as jnp
from jax import lax
from jax.experimental import pallas as pl
from jax.experimental.pallas import tpu as pltpu
from jax.experimental.pallas import tpu_sc as plsc

NROW = 32
NCOL = 1_000_000
NSAMP = 64
BC = 128
SUB = 8
CB = SUB * BC
NCB = 128
CC = NCB * CB
NSTEP = -(-NCOL // CC)
NG = NROW // 8
GI = NG * NSTEP
NB = -(-NCOL // CB)
NBP = NSTEP * NCB
MAXA = (NCOL // BC) * BC
L = 16


def _pass1_body(x_ref, bs_ref):
    i = pl.program_id(1)

    def emit(mask_cols):
        for b in range(NCB):
            acc = x_ref[:, b * CB:b * CB + BC]
            for j in range(1, SUB):
                sl = x_ref[:, b * CB + j * BC:b * CB + (j + 1) * BC]
                if mask_cols:
                    col = (i * CC + b * CB + j * BC
                           + lax.broadcasted_iota(jnp.int32, (8, BC), 1))
                    sl = jnp.where(col < NCOL, sl, 0.0)
                acc = acc + sl
            if mask_cols:
                col = (i * CC + b * CB
                       + lax.broadcasted_iota(jnp.int32, (8, BC), 1))
                acc = jnp.where(col < NCOL, acc, 0.0)
            bs_ref[0, :, b:b + 1] = jnp.sum(acc, axis=-1, keepdims=True)

    @pl.when(i < NSTEP - 1)
    def _full():
        emit(False)

    @pl.when(i == NSTEP - 1)
    def _masked():
        emit(True)


def _pass1(x):
    return pl.pallas_call(
        _pass1_body,
        grid=(NG, NSTEP),
        in_specs=[pl.BlockSpec((8, CC), lambda g, i: (g, i))],
        out_specs=pl.BlockSpec((1, 8, NCB), lambda g, i: (g * NSTEP + i, 0, 0)),
        out_shape=jax.ShapeDtypeStruct((GI, 8, NCB), jnp.float32),
    )(x)


def _sc_compiler_params():
    cp = pltpu.CompilerParams(use_tc_tiling_on_sc=True)
    if "needs_layout_passes" in pltpu.CompilerParams.__dataclass_fields__:
        cp = dataclasses.replace(cp, needs_layout_passes=False)
    return cp


def _sample_body(xl_hbm, bs_hbm, u_hbm, o_hbm, bs_v, bcum_v, t_v, blk_v,
                 res_v, sem):
    r = lax.axis_index("s") * 2 + lax.axis_index("c")
    g = r // 8
    s = r % 8
    iota = lax.iota(jnp.int32, L)

    copies = [
        pltpu.async_copy(bs_hbm.at[g * NSTEP + i, s],
                         bs_v.at[pl.ds(i * NCB, NCB)], sem)
        for i in range(NSTEP)
    ]
    for cp in copies:
        cp.wait()

    def cum_body(i, carry):
        c = plsc.cumsum(bs_v[pl.ds(i * L, L)]) + carry
        bcum_v[pl.ds(i * L, L)] = c
        return jnp.max(c)

    total = lax.fori_loop(0, NBP // L, cum_body, jnp.float32(0.0))

    pltpu.sync_copy(u_hbm.at[r], t_v)
    for j in range(NSAMP // L):
        t_v[pl.ds(j * L, L)] = t_v[pl.ds(j * L, L)] * total

    def sample_body(k, acc_):
        kk = jnp.full((L,), k, jnp.int32)
        tb = plsc.load_gather(t_v, [kk])
        t_s = jnp.max(tb)

        lo = jnp.int32(0)
        basev = jnp.zeros((L,), jnp.float32)
        for st in (64, 4, 1):
            p = jnp.minimum(lo + (iota + 1) * st - 1, NBP - 1)
            v = plsc.load_gather(bcum_v, [p])
            le = v <= tb
            cnt = jnp.sum(le.astype(jnp.int32))
            basev = jnp.maximum(basev, jnp.where(le, v, 0.0))
            lo = lo + cnt * st

        block = jnp.minimum(lo, NB - 1)
        base = jnp.max(basev)
        a = block * CB

        cps = [
            pltpu.async_copy(
                xl_hbm.at[r, pl.ds(jnp.minimum(a + j * BC, MAXA), BC)],
                blk_v.at[pl.ds(j * BC, BC)], sem)
            for j in range(SUB)
        ]
        for cp in cps:
            cp.wait()

        pre = base
        nfull = jnp.int32(0)
        fbase = base
        for j in range(SUB):
            acc = blk_v[pl.ds(j * BC, L)]
            for gg in range(1, BC // L):
                acc = acc + blk_v[pl.ds(j * BC + gg * L, L)]
            ok = (a + j * BC) < NCOL
            pre2 = pre + jnp.where(ok, jnp.sum(acc), 0.0)
            lt = pre2 <= t_s
            nfull = nfull + jnp.where(lt, 1, 0)
            fbase = jnp.where(lt, pre2, fbase)
            pre = pre2

        off = jnp.minimum(nfull, SUB - 1) * BC
        cnt2 = jnp.int32(0)
        carry = fbase
        for gg in range(BC // L):
            cs = plsc.cumsum(blk_v[pl.ds(off + gg * L, L)]) + carry
            cnt2 = cnt2 + jnp.sum((cs <= tb).astype(jnp.int32))
            carry = jnp.max(cs)

        final = jnp.minimum(a + nfull * BC + cnt2, NCOL - 1)
        plsc.store_scatter(res_v, [kk], jnp.full((L,), final, jnp.int32),
                           mask=iota == 0)
        return acc_

    lax.fori_loop(0, NSAMP, sample_body, jnp.int32(0))
    pltpu.sync_copy(res_v, o_hbm.at[r])


@functools.partial(
    pl.kernel,
    out_type=jax.ShapeDtypeStruct((NROW, NSAMP), jnp.int32),
    mesh=plsc.VectorSubcoreMesh(core_axis_name="c", subcore_axis_name="s"),
    scratch_types=[
        pltpu.VMEM((NBP,), jnp.float32),
        pltpu.VMEM((NBP,), jnp.float32),
        pltpu.VMEM((NSAMP,), jnp.float32),
        pltpu.VMEM((SUB * BC,), jnp.float32),
        pltpu.VMEM((NSAMP,), jnp.int32),
        pltpu.SemaphoreType.DMA,
    ],
    compiler_params=_sc_compiler_params(),
)
def _sample_kernel(xl_hbm, bs_hbm, u_hbm, o_hbm, bs_v, bcum_v, t_v, blk_v,
                   res_v, sem):
    _sample_body(xl_hbm, bs_hbm, u_hbm, o_hbm, bs_v, bcum_v, t_v, blk_v,
                 res_v, sem)


def kernel(x):
    bs = _pass1(x)
    u = jax.random.uniform(jax.random.key(42), (NROW, NSAMP),
                           dtype=jnp.float32)
    idx = _sample_kernel(x, bs, u)
    return idx.astype(jnp.int64)


# speedup vs baseline: 33.6896x; 1.2712x over previous
import dataclasses
import functools

import jax
import jax.numpy as jnp
from jax import lax
from jax.experimental import pallas as pl
from jax.experimental.pallas import tpu as pltpu
from jax.experimental.pallas import tpu_sc as plsc

NROW = 32
NCOL = 1_000_000
NSAMP = 64
BC = 128
SUB = 8
CB = SUB * BC
NCB = 128
CC = NCB * CB
NSTEP = -(-NCOL // CC)
NG = NROW // 8
GI = NG * NSTEP
NB = -(-NCOL // CB)
NBP = NSTEP * NCB
MAXA = (NCOL // BC) * BC
L = 16


def _pass1_body(x_ref, bs_ref):
    i = pl.program_id(1)

    def emit(mask_cols):
        for b in range(NCB):
            acc = x_ref[:, b * CB:b * CB + BC]
            for j in range(1, SUB):
                sl = x_ref[:, b * CB + j * BC:b * CB + (j + 1) * BC]
                if mask_cols:
                    col = (i * CC + b * CB + j * BC
                           + lax.broadcasted_iota(jnp.int32, (8, BC), 1))
                    sl = jnp.where(col < NCOL, sl, 0.0)
                acc = acc + sl
            if mask_cols:
                col = (i * CC + b * CB
                       + lax.broadcasted_iota(jnp.int32, (8, BC), 1))
                acc = jnp.where(col < NCOL, acc, 0.0)
            bs_ref[0, :, b:b + 1] = jnp.sum(acc, axis=-1, keepdims=True)

    @pl.when(i < NSTEP - 1)
    def _full():
        emit(False)

    @pl.when(i == NSTEP - 1)
    def _masked():
        emit(True)


def _pass1(x):
    return pl.pallas_call(
        _pass1_body,
        grid=(NG, NSTEP),
        in_specs=[pl.BlockSpec((8, CC), lambda g, i: (g, i))],
        out_specs=pl.BlockSpec((1, 8, NCB), lambda g, i: (g * NSTEP + i, 0, 0)),
        out_shape=jax.ShapeDtypeStruct((GI, 8, NCB), jnp.float32),
    )(x)


def _sc_compiler_params():
    cp = pltpu.CompilerParams(use_tc_tiling_on_sc=True)
    if "needs_layout_passes" in pltpu.CompilerParams.__dataclass_fields__:
        cp = dataclasses.replace(cp, needs_layout_passes=False)
    return cp


def _sample_body(xl_hbm, bs_hbm, u_hbm, o_hbm, bs_v, bcum_v, t_v, blk_v,
                 res_v, sem):
    r = lax.axis_index("s") * 2 + lax.axis_index("c")
    g = r // 8
    s = r % 8
    iota = lax.iota(jnp.int32, L)

    copies = [
        pltpu.async_copy(bs_hbm.at[g * NSTEP + i, s],
                         bs_v.at[pl.ds(i * NCB, NCB)], sem)
        for i in range(NSTEP)
    ]
    for cp in copies:
        cp.wait()

    def cum_body(i, carry):
        c = plsc.cumsum(bs_v[pl.ds(i * L, L)]) + carry
        bcum_v[pl.ds(i * L, L)] = c
        return jnp.max(c)

    total = lax.fori_loop(0, NBP // L, cum_body, jnp.float32(0.0))

    pltpu.sync_copy(u_hbm.at[r], t_v)
    for j in range(NSAMP // L):
        t_v[pl.ds(j * L, L)] = t_v[pl.ds(j * L, L)] * total

    def search(k):
        kk = jnp.full((L,), k, jnp.int32)
        tb = plsc.load_gather(t_v, [kk])
        lo = jnp.int32(0)
        basev = jnp.zeros((L,), jnp.float32)
        for st in (64, 4, 1):
            p = jnp.minimum(lo + (iota + 1) * st - 1, NBP - 1)
            v = plsc.load_gather(bcum_v, [p])
            le = v <= tb
            cnt = jnp.sum(le.astype(jnp.int32))
            basev = jnp.maximum(basev, jnp.where(le, v, 0.0))
            lo = lo + cnt * st
        return jnp.minimum(lo, NB - 1), jnp.max(basev), jnp.max(tb)

    def issue(par, block):
        a = block * CB
        for j in range(SUB):
            pltpu.async_copy(
                xl_hbm.at[r, pl.ds(jnp.minimum(a + j * BC, MAXA), BC)],
                blk_v.at[pl.ds(par + j * BC, BC)], sem)

    def drain(par):
        for j in range(SUB):
            pltpu.make_async_copy(
                xl_hbm.at[r, pl.ds(0, BC)],
                blk_v.at[pl.ds(par + j * BC, BC)], sem).wait()

    def sample_body(k, carry):
        block, base, t_s = carry
        nblock, nbase, nt = search(jnp.minimum(k + 1, NSAMP - 1))
        issue(lax.rem(k + 1, 2) * CB, nblock)
        par = lax.rem(k, 2) * CB
        drain(par)

        tb = jnp.full((L,), t_s)
        a = block * CB
        pre = base
        nfull = jnp.int32(0)
        fbase = base
        for j in range(SUB):
            acc = blk_v[pl.ds(par + j * BC, L)]
            for gg in range(1, BC // L):
                acc = acc + blk_v[pl.ds(par + j * BC + gg * L, L)]
            ok = (a + j * BC) < NCOL
            pre2 = pre + jnp.where(ok, jnp.sum(acc), 0.0)
            lt = pre2 <= t_s
            nfull = nfull + jnp.where(lt, 1, 0)
            fbase = jnp.where(lt, pre2, fbase)
            pre = pre2

        off = par + jnp.minimum(nfull, SUB - 1) * BC
        cnt2 = jnp.int32(0)
        carry2 = fbase
        for gg in range(BC // L):
            cs = plsc.cumsum(blk_v[pl.ds(off + gg * L, L)]) + carry2
            cnt2 = cnt2 + jnp.sum((cs <= tb).astype(jnp.int32))
            carry2 = jnp.max(cs)

        final = jnp.minimum(a + nfull * BC + cnt2, NCOL - 1)
        plsc.store_scatter(res_v, [jnp.full((L,), k, jnp.int32)],
                           jnp.full((L,), final, jnp.int32),
                           mask=iota == 0)
        return nblock, nbase, nt

    b0, base0, t0 = search(jnp.int32(0))
    issue(jnp.int32(0), b0)
    lax.fori_loop(0, NSAMP, sample_body, (b0, base0, t0))
    drain(jnp.int32(0))
    pltpu.sync_copy(res_v, o_hbm.at[r])


@functools.partial(
    pl.kernel,
    out_type=jax.ShapeDtypeStruct((NROW, NSAMP), jnp.int32),
    mesh=plsc.VectorSubcoreMesh(core_axis_name="c", subcore_axis_name="s"),
    scratch_types=[
        pltpu.VMEM((NBP,), jnp.float32),
        pltpu.VMEM((NBP,), jnp.float32),
        pltpu.VMEM((NSAMP,), jnp.float32),
        pltpu.VMEM((2 * SUB * BC,), jnp.float32),
        pltpu.VMEM((NSAMP,), jnp.int32),
        pltpu.SemaphoreType.DMA,
    ],
    compiler_params=_sc_compiler_params(),
)
def _sample_kernel(xl_hbm, bs_hbm, u_hbm, o_hbm, bs_v, bcum_v, t_v, blk_v,
                   res_v, sem):
    _sample_body(xl_hbm, bs_hbm, u_hbm, o_hbm, bs_v, bcum_v, t_v, blk_v,
                 res_v, sem)


def kernel(x):
    bs = _pass1(x)
    u = jax.random.uniform(jax.random.key(42), (NROW, NSAMP),
                           dtype=jnp.float32)
    idx = _sample_kernel(x, bs, u)
    return idx.astype(jnp.int64)
